# baseline (speedup 1.0000x reference)
import jax
import jax.numpy as jnp
from jax import lax
from jax.experimental import pallas as pl
from jax.experimental.pallas import tpu as pltpu
from jax.experimental.pallas import tpu_sc as plsc

N_NODES = 100000
B_SAMPLES = 64

_LANES = 16
_CHUNK = 3136
_VECS = _CHUNK // _LANES


def _sc_body(ft, lat_hbm, bid_hbm, ot,
             table_v, bid_v, fx_v, fy_v, fz_v, ox_v, oy_v, oz_v):
    wid = lax.axis_index("s") * 2 + lax.axis_index("c")
    base = lax.min(wid * _CHUNK, N_NODES - _CHUNK)

    pltpu.sync_copy(lat_hbm, table_v)
    pltpu.sync_copy(bid_hbm.at[pl.ds(base, _CHUNK)], bid_v)
    pltpu.sync_copy(ft.at[pl.ds(base, _CHUNK)], fx_v)
    pltpu.sync_copy(ft.at[pl.ds(N_NODES + base, _CHUNK)], fy_v)
    pltpu.sync_copy(ft.at[pl.ds(2 * N_NODES + base, _CHUNK)], fz_v)

    @plsc.parallel_loop(0, _VECS, unroll=8)
    def step(i):
        sl = pl.ds(i * _LANES, _LANES)
        b9 = bid_v[sl] * 9
        f0 = fx_v[sl]
        f1 = fy_v[sl]
        f2 = fz_v[sl]
        o_refs = (ox_v, oy_v, oz_v)
        for k in range(3):
            a0 = plsc.load_gather(table_v, [b9 + k])
            a1 = plsc.load_gather(table_v, [b9 + (3 + k)])
            a2 = plsc.load_gather(table_v, [b9 + (6 + k)])
            o_refs[k][sl] = f0 * a0 + f1 * a1 + f2 * a2

    pltpu.sync_copy(ox_v, ot.at[pl.ds(base, _CHUNK)])
    pltpu.sync_copy(oy_v, ot.at[pl.ds(N_NODES + base, _CHUNK)])
    pltpu.sync_copy(oz_v, ot.at[pl.ds(2 * N_NODES + base, _CHUNK)])


@jax.jit
def _run(frac_coords, lattice_matrices, batch_id):
    mesh = plsc.VectorSubcoreMesh(core_axis_name="c", subcore_axis_name="s")
    ot = pl.kernel(
        _sc_body,
        out_type=jax.ShapeDtypeStruct((3 * N_NODES,), jnp.float32),
        mesh=mesh,
        scratch_types=[
            pltpu.VMEM((B_SAMPLES * 9,), jnp.float32),
            pltpu.VMEM((_CHUNK,), jnp.int32),
        ] + [pltpu.VMEM((_CHUNK,), jnp.float32)] * 6,
        compiler_params=pltpu.CompilerParams(needs_layout_passes=False),
    )(frac_coords.T.reshape(-1),
      lattice_matrices.reshape(-1).astype(jnp.float32),
      batch_id.astype(jnp.int32))
    return ot.reshape(3, N_NODES).T


def kernel(frac_coords, lattice_matrices, batch_id):
    return _run(frac_coords, lattice_matrices, batch_id)


# speedup vs baseline: 1.1117x; 1.1117x over previous
import jax
import jax.numpy as jnp
from jax import lax
from jax.experimental import pallas as pl
from jax.experimental.pallas import tpu as pltpu
from jax.experimental.pallas import tpu_sc as plsc

N_NODES = 100000
B_SAMPLES = 64

_LANES = 16
_CHUNK = 3136
_VECS = _CHUNK // _LANES


def _sc_body(ft, lat_hbm, bid_hbm, ot,
             table_v, bid_v, fx_v, fy_v, fz_v, ox_v, oy_v, oz_v, sem):
    wid = lax.axis_index("s") * 2 + lax.axis_index("c")
    base = lax.min(wid * _CHUNK, N_NODES - _CHUNK)

    cps = [
        pltpu.async_copy(lat_hbm, table_v, sem),
        pltpu.async_copy(bid_hbm.at[pl.ds(base, _CHUNK)], bid_v, sem),
        pltpu.async_copy(ft.at[pl.ds(base, _CHUNK)], fx_v, sem),
        pltpu.async_copy(ft.at[pl.ds(N_NODES + base, _CHUNK)], fy_v, sem),
        pltpu.async_copy(ft.at[pl.ds(2 * N_NODES + base, _CHUNK)], fz_v, sem),
    ]
    for cp in cps:
        cp.wait()

    @plsc.parallel_loop(0, _VECS, unroll=4)
    def step(i):
        sl = pl.ds(i * _LANES, _LANES)
        b9 = bid_v[sl] * 9
        f0 = fx_v[sl]
        f1 = fy_v[sl]
        f2 = fz_v[sl]
        o_refs = (ox_v, oy_v, oz_v)
        for k in range(3):
            a0 = plsc.load_gather(table_v, [b9 + k])
            a1 = plsc.load_gather(table_v, [b9 + (3 + k)])
            a2 = plsc.load_gather(table_v, [b9 + (6 + k)])
            o_refs[k][sl] = f0 * a0 + f1 * a1 + f2 * a2

    pltpu.sync_copy(ox_v, ot.at[pl.ds(base, _CHUNK)])
    pltpu.sync_copy(oy_v, ot.at[pl.ds(N_NODES + base, _CHUNK)])
    pltpu.sync_copy(oz_v, ot.at[pl.ds(2 * N_NODES + base, _CHUNK)])


@jax.jit
def _run(frac_coords, lattice_matrices, batch_id):
    mesh = plsc.VectorSubcoreMesh(core_axis_name="c", subcore_axis_name="s")
    ot = pl.kernel(
        _sc_body,
        out_type=jax.ShapeDtypeStruct((3 * N_NODES,), jnp.float32),
        mesh=mesh,
        scratch_types=[
            pltpu.VMEM((B_SAMPLES * 9,), jnp.float32),
            pltpu.VMEM((_CHUNK,), jnp.int32),
        ] + [pltpu.VMEM((_CHUNK,), jnp.float32)] * 6
          + [pltpu.SemaphoreType.DMA],
        compiler_params=pltpu.CompilerParams(needs_layout_passes=False),
    )(frac_coords.T.reshape(-1),
      lattice_matrices.reshape(-1).astype(jnp.float32),
      batch_id.astype(jnp.int32))
    return ot.reshape(3, N_NODES).T


def kernel(frac_coords, lattice_matrices, batch_id):
    return _run(frac_coords, lattice_matrices, batch_id)


# speedup vs baseline: 1.1191x; 1.0067x over previous
import jax
import jax.numpy as jnp
from jax import lax
from jax.experimental import pallas as pl
from jax.experimental.pallas import tpu as pltpu
from jax.experimental.pallas import tpu_sc as plsc

N_NODES = 100000
B_SAMPLES = 64

_LANES = 16
_CHUNK = 3136
_VECS = _CHUNK // _LANES


def _sc_body(ft, lat_hbm, bid_hbm, ot,
             table_v, bid_v, fx_v, fy_v, fz_v, ox_v, oy_v, oz_v, sem):
    wid = lax.axis_index("s") * 2 + lax.axis_index("c")
    base = lax.min(wid * _CHUNK, N_NODES - _CHUNK)

    cps = [
        pltpu.async_copy(lat_hbm, table_v, sem),
        pltpu.async_copy(bid_hbm.at[pl.ds(base, _CHUNK)], bid_v, sem),
        pltpu.async_copy(ft.at[pl.ds(base, _CHUNK)], fx_v, sem),
        pltpu.async_copy(ft.at[pl.ds(N_NODES + base, _CHUNK)], fy_v, sem),
        pltpu.async_copy(ft.at[pl.ds(2 * N_NODES + base, _CHUNK)], fz_v, sem),
    ]
    for cp in cps:
        cp.wait()

    @plsc.parallel_loop(0, _VECS, unroll=4)
    def step(i):
        sl = pl.ds(i * _LANES, _LANES)
        b9 = bid_v[sl] * 9
        f0 = fx_v[sl]
        f1 = fy_v[sl]
        f2 = fz_v[sl]
        o_refs = (ox_v, oy_v, oz_v)
        for k in range(3):
            a0 = plsc.load_gather(table_v, [b9 + k])
            a1 = plsc.load_gather(table_v, [b9 + (3 + k)])
            a2 = plsc.load_gather(table_v, [b9 + (6 + k)])
            o_refs[k][sl] = f0 * a0 + f1 * a1 + f2 * a2

    ocps = [
        pltpu.async_copy(ox_v, ot.at[pl.ds(base, _CHUNK)], sem),
        pltpu.async_copy(oy_v, ot.at[pl.ds(N_NODES + base, _CHUNK)], sem),
        pltpu.async_copy(oz_v, ot.at[pl.ds(2 * N_NODES + base, _CHUNK)], sem),
    ]
    for cp in ocps:
        cp.wait()


@jax.jit
def _run(frac_coords, lattice_matrices, batch_id):
    mesh = plsc.VectorSubcoreMesh(core_axis_name="c", subcore_axis_name="s")
    ot = pl.kernel(
        _sc_body,
        out_type=jax.ShapeDtypeStruct((3 * N_NODES,), jnp.float32),
        mesh=mesh,
        scratch_types=[
            pltpu.VMEM((B_SAMPLES * 9,), jnp.float32),
            pltpu.VMEM((_CHUNK,), jnp.int32),
        ] + [pltpu.VMEM((_CHUNK,), jnp.float32)] * 6
          + [pltpu.SemaphoreType.DMA],
        compiler_params=pltpu.CompilerParams(needs_layout_passes=False),
    )(frac_coords.T.reshape(-1),
      lattice_matrices.reshape(-1).astype(jnp.float32),
      batch_id.astype(jnp.int32))
    return ot.reshape(3, N_NODES).T


def kernel(frac_coords, lattice_matrices, batch_id):
    return _run(frac_coords, lattice_matrices, batch_id)
